# Initial kernel scaffold; baseline (speedup 1.0000x reference)
import jax
import jax.numpy as jnp
from jax.experimental import pallas as pl


def kernel(x1, edge_index1, batch1, x2, edge_index2, batch2, W1, b1, g1, be1, W2, b2, g2, be2, W3, b3, g3, be3, Wf, bf, Wc1, bc1, Wc2, bc2):
    raise NotImplementedError("write your pallas kernel here")


# speedup vs baseline: 4.5567x; 4.5567x over previous
import functools

import jax
import jax.numpy as jnp
from jax import lax
from jax.experimental import pallas as pl
from jax.experimental.pallas import tpu as pltpu
from jax.experimental.pallas import tpu_sc as plsc

_N = 10000
_E = 320000
_D = 128
_G = 64
_NW = 32
_C = 128
_NCHUNK = _E // _C
_CPW = -(-_NCHUNK // _NW)
_RPT = 632
_RPT_LAST = _N - 15 * _RPT


def _segsum_sc(h, src, dst, zeros):
    mesh = plsc.VectorSubcoreMesh(core_axis_name="c", subcore_axis_name="s")

    @functools.partial(
        pl.kernel,
        out_type=jax.ShapeDtypeStruct((2, _N, _D), jnp.float32),
        mesh=mesh,
        scratch_types=[
            pltpu.VMEM((_C,), jnp.int32),
            pltpu.VMEM((_C,), jnp.int32),
            pltpu.VMEM((_C, _D), jnp.float32),
            pltpu.VMEM_SHARED((_N, _D), jnp.float32),
            pltpu.SemaphoreType.DMA,
        ],
    )
    def seg_kernel(h_hbm, src_hbm, dst_hbm, zero_hbm, out_hbm,
                   sidx, didx, rows, shared, sem):
        cid = lax.axis_index("c")
        sid = lax.axis_index("s")
        w = sid * 2 + cid

        def for_my_rows(fn):
            @pl.when(sid < 15)
            def _():
                fn(pl.ds(sid * _RPT, _RPT))

            @pl.when(sid == 15)
            def _():
                fn(pl.ds(15 * _RPT, _RPT_LAST))

        def init_rows(rslice):
            @pl.when(cid == 0)
            def _():
                pltpu.sync_copy(h_hbm.at[rslice], shared.at[rslice])

            @pl.when(cid != 0)
            def _():
                pltpu.sync_copy(zero_hbm.at[rslice], shared.at[rslice])

        for_my_rows(init_rows)
        plsc.subcore_barrier()

        @pl.loop(0, _CPW)
        def _(i):
            chunk = w + i * _NW

            @pl.when(chunk < _NCHUNK)
            def _():
                base = chunk * _C
                pltpu.sync_copy(src_hbm.at[pl.ds(base, _C)], sidx)
                pltpu.sync_copy(dst_hbm.at[pl.ds(base, _C)], didx)
                pltpu.async_copy(h_hbm.at[sidx], rows, sem).wait()
                pltpu.sync_copy(rows, shared.at[didx], add=True)

        plsc.subcore_barrier()
        for_my_rows(
            lambda rslice: pltpu.sync_copy(shared.at[rslice],
                                           out_hbm.at[cid].at[rslice]))

    return seg_kernel(h, src, dst, zeros)


def _layer_body(p_ref, w_ref, b_ref, g_ref, be_ref, o_ref):
    a = p_ref[0] + p_ref[1]
    pre = lax.dot_general(
        a, w_ref[...], (((1,), (0,)), ((), ())),
        precision=lax.Precision.HIGHEST,
        preferred_element_type=jnp.float32,
    ) + b_ref[...][None, :]
    mu = jnp.mean(pre, axis=0, keepdims=True)
    var = jnp.mean((pre - mu) ** 2, axis=0, keepdims=True)
    hn = (pre - mu) / jnp.sqrt(var + 1e-5) * g_ref[...][None, :] + be_ref[...][None, :]
    o_ref[...] = jnp.maximum(hn, 0.0)


def _tc_layer(parts, W, b, g, be):
    return pl.pallas_call(
        _layer_body,
        out_shape=jax.ShapeDtypeStruct((_N, _D), jnp.float32),
    )(parts, W, b, g, be)


def _head_body(h1_ref, bat1_ref, h2_ref, bat2_ref, wf_ref, bf_ref,
               wc1_ref, bc1_ref, wc2_ref, bc2_ref, o_ref):
    def pooled_emb(h_ref, bat_ref):
        bat = bat_ref[...]
        gids = lax.broadcasted_iota(jnp.int32, (_G, _N), 0)
        m = (bat[None, :] == gids).astype(jnp.float32)
        sums = lax.dot_general(
            m, h_ref[...], (((1,), (0,)), ((), ())),
            precision=lax.Precision.HIGHEST,
            preferred_element_type=jnp.float32,
        )
        cnt = jnp.sum(m, axis=1, keepdims=True)
        pooled = sums / jnp.maximum(cnt, 1.0)
        return lax.dot_general(
            pooled, wf_ref[...], (((1,), (0,)), ((), ())),
            precision=lax.Precision.HIGHEST,
            preferred_element_type=jnp.float32,
        ) + bf_ref[...][None, :]

    v1 = pooled_emb(h1_ref, bat1_ref)
    v2 = pooled_emb(h2_ref, bat2_ref)
    d = jnp.abs(v1 - v2)
    z = lax.dot_general(
        d, wc1_ref[...], (((1,), (0,)), ((), ())),
        precision=lax.Precision.HIGHEST,
        preferred_element_type=jnp.float32,
    ) + bc1_ref[...][None, :]
    z = jnp.maximum(z, 0.0)
    s = jnp.sum(z * wc2_ref[...][:, 0][None, :], axis=1, keepdims=True)
    s = s + bc2_ref[...][None, :]
    o_ref[...] = jax.nn.sigmoid(s)


def _head(h1, bat1, h2, bat2, Wf, bf, Wc1, bc1, Wc2, bc2):
    return pl.pallas_call(
        _head_body,
        out_shape=jax.ShapeDtypeStruct((_G, 1), jnp.float32),
    )(h1, bat1, h2, bat2, Wf, bf, Wc1, bc1, Wc2, bc2)


def kernel(x1, edge_index1, batch1, x2, edge_index2, batch2,
           W1, b1, g1, be1, W2, b2, g2, be2, W3, b3, g3, be3,
           Wf, bf, Wc1, bc1, Wc2, bc2):
    zeros = jnp.zeros((_N, _D), jnp.float32)
    layer_weights = ((W1, b1, g1, be1), (W2, b2, g2, be2), (W3, b3, g3, be3))

    def enc(x, ei):
        src, dst = ei[0], ei[1]
        h = x
        for (W, b, g, be) in layer_weights:
            parts = _segsum_sc(h, src, dst, zeros)
            h = _tc_layer(parts, W, b, g, be)
        return h

    h1 = enc(x1, edge_index1)
    h2 = enc(x2, edge_index2)
    return _head(h1, batch1, h2, batch2, Wf, bf, Wc1, bc1, Wc2, bc2)


# speedup vs baseline: 6.0170x; 1.3205x over previous
import functools

import jax
import jax.numpy as jnp
from jax import lax
from jax.experimental import pallas as pl
from jax.experimental.pallas import tpu as pltpu
from jax.experimental.pallas import tpu_sc as plsc

_N = 10000
_E = 320000
_D = 128
_G = 64
_NW = 32
_EPW = _E // _NW
_C = 80
_NSC = _EPW // _C
_RPT = 632
_RPT_LAST = _N - 15 * _RPT


def _segsum_sc(h, src, dst, zeros):
    mesh = plsc.VectorSubcoreMesh(core_axis_name="c", subcore_axis_name="s")

    @functools.partial(
        pl.kernel,
        out_type=jax.ShapeDtypeStruct((2, _N, _D), jnp.float32),
        mesh=mesh,
        scratch_types=[
            pltpu.VMEM((_C,), jnp.int32),
            pltpu.VMEM((_C,), jnp.int32),
            pltpu.VMEM((_C, _D), jnp.float32),
            pltpu.VMEM((_C,), jnp.int32),
            pltpu.VMEM((_C,), jnp.int32),
            pltpu.VMEM((_C, _D), jnp.float32),
            pltpu.VMEM_SHARED((_N, _D), jnp.float32),
            pltpu.SemaphoreType.DMA,
            pltpu.SemaphoreType.DMA,
            pltpu.SemaphoreType.DMA,
            pltpu.SemaphoreType.DMA,
        ],
    )
    def seg_kernel(h_hbm, src_hbm, dst_hbm, zero_hbm, out_hbm,
                   sidx0, didx0, rows0, sidx1, didx1, rows1, shared,
                   gsem0, gsem1, ssem0, ssem1):
        cid = lax.axis_index("c")
        sid = lax.axis_index("s")
        w = sid * 2 + cid

        def for_my_rows(fn):
            @pl.when(sid < 15)
            def _():
                fn(pl.ds(sid * _RPT, _RPT))

            @pl.when(sid == 15)
            def _():
                fn(pl.ds(15 * _RPT, _RPT_LAST))

        def init_rows(rslice):
            @pl.when(cid == 0)
            def _():
                pltpu.sync_copy(h_hbm.at[rslice], shared.at[rslice])

            @pl.when(cid != 0)
            def _():
                pltpu.sync_copy(zero_hbm.at[rslice], shared.at[rslice])

        for_my_rows(init_rows)
        plsc.subcore_barrier()

        ebase = w * _EPW
        bufs = ((sidx0, didx0, rows0, gsem0, ssem0),
                (sidx1, didx1, rows1, gsem1, ssem1))

        def idx_copy(i, b):
            base = ebase + i * _C
            pltpu.sync_copy(src_hbm.at[pl.ds(base, _C)], b[0])
            pltpu.sync_copy(dst_hbm.at[pl.ds(base, _C)], b[1])

        def gstart(b):
            pltpu.async_copy(h_hbm.at[b[0]], b[2], b[3])

        def gwait(b):
            pltpu.make_async_copy(h_hbm.at[b[0]], b[2], b[3]).wait()

        def sstart(b):
            pltpu.async_copy(b[2], shared.at[b[1]], b[4], add=True)

        def swait(b):
            pltpu.make_async_copy(b[2], shared.at[b[1]], b[4]).wait()

        idx_copy(0, bufs[0])
        gstart(bufs[0])

        @pl.loop(0, _NSC - 1, step=2)
        def _(i):
            @pl.when(i >= 2)
            def _():
                swait(bufs[1])

            idx_copy(i + 1, bufs[1])
            gwait(bufs[0])
            gstart(bufs[1])
            sstart(bufs[0])

            swait(bufs[0])

            @pl.when(i + 2 < _NSC)
            def _():
                idx_copy(i + 2, bufs[0])

            gwait(bufs[1])

            @pl.when(i + 2 < _NSC)
            def _():
                gstart(bufs[0])

            sstart(bufs[1])

        gwait(bufs[0])
        swait(bufs[1])
        sstart(bufs[0])
        swait(bufs[0])

        plsc.subcore_barrier()
        for_my_rows(
            lambda rslice: pltpu.sync_copy(shared.at[rslice],
                                           out_hbm.at[cid].at[rslice]))

    return seg_kernel(h, src, dst, zeros)


def _layer_body(p_ref, w_ref, b_ref, g_ref, be_ref, o_ref):
    a = p_ref[0] + p_ref[1]
    pre = lax.dot_general(
        a, w_ref[...], (((1,), (0,)), ((), ())),
        precision=lax.Precision.HIGHEST,
        preferred_element_type=jnp.float32,
    ) + b_ref[...][None, :]
    mu = jnp.mean(pre, axis=0, keepdims=True)
    var = jnp.mean((pre - mu) ** 2, axis=0, keepdims=True)
    hn = (pre - mu) / jnp.sqrt(var + 1e-5) * g_ref[...][None, :] + be_ref[...][None, :]
    o_ref[...] = jnp.maximum(hn, 0.0)


def _tc_layer(parts, W, b, g, be):
    return pl.pallas_call(
        _layer_body,
        out_shape=jax.ShapeDtypeStruct((_N, _D), jnp.float32),
    )(parts, W, b, g, be)


def _head_body(h1_ref, bat1_ref, h2_ref, bat2_ref, wf_ref, bf_ref,
               wc1_ref, bc1_ref, wc2_ref, bc2_ref, o_ref):
    def pooled_emb(h_ref, bat_ref):
        bat = bat_ref[...]
        gids = lax.broadcasted_iota(jnp.int32, (_G, _N), 0)
        m = (bat[None, :] == gids).astype(jnp.float32)
        sums = lax.dot_general(
            m, h_ref[...], (((1,), (0,)), ((), ())),
            precision=lax.Precision.HIGHEST,
            preferred_element_type=jnp.float32,
        )
        cnt = jnp.sum(m, axis=1, keepdims=True)
        pooled = sums / jnp.maximum(cnt, 1.0)
        return lax.dot_general(
            pooled, wf_ref[...], (((1,), (0,)), ((), ())),
            precision=lax.Precision.HIGHEST,
            preferred_element_type=jnp.float32,
        ) + bf_ref[...][None, :]

    v1 = pooled_emb(h1_ref, bat1_ref)
    v2 = pooled_emb(h2_ref, bat2_ref)
    d = jnp.abs(v1 - v2)
    z = lax.dot_general(
        d, wc1_ref[...], (((1,), (0,)), ((), ())),
        precision=lax.Precision.HIGHEST,
        preferred_element_type=jnp.float32,
    ) + bc1_ref[...][None, :]
    z = jnp.maximum(z, 0.0)
    s = jnp.sum(z * wc2_ref[...][:, 0][None, :], axis=1, keepdims=True)
    s = s + bc2_ref[...][None, :]
    o_ref[...] = jax.nn.sigmoid(s)


def _head(h1, bat1, h2, bat2, Wf, bf, Wc1, bc1, Wc2, bc2):
    return pl.pallas_call(
        _head_body,
        out_shape=jax.ShapeDtypeStruct((_G, 1), jnp.float32),
    )(h1, bat1, h2, bat2, Wf, bf, Wc1, bc1, Wc2, bc2)


def kernel(x1, edge_index1, batch1, x2, edge_index2, batch2,
           W1, b1, g1, be1, W2, b2, g2, be2, W3, b3, g3, be3,
           Wf, bf, Wc1, bc1, Wc2, bc2):
    zeros = jnp.zeros((_N, _D), jnp.float32)
    layer_weights = ((W1, b1, g1, be1), (W2, b2, g2, be2), (W3, b3, g3, be3))

    def enc(x, ei):
        src, dst = ei[0], ei[1]
        h = x
        for (W, b, g, be) in layer_weights:
            parts = _segsum_sc(h, src, dst, zeros)
            h = _tc_layer(parts, W, b, g, be)
        return h

    h1 = enc(x1, edge_index1)
    h2 = enc(x2, edge_index2)
    return _head(h1, batch1, h2, batch2, Wf, bf, Wc1, bc1, Wc2, bc2)


# speedup vs baseline: 7.9900x; 1.3279x over previous
import functools

import jax
import jax.numpy as jnp
from jax import lax
from jax.experimental import pallas as pl
from jax.experimental.pallas import tpu as pltpu
from jax.experimental.pallas import tpu_sc as plsc

_N = 10000
_E = 320000
_D = 128
_G = 64
_NW = 32
_EPW = _E // _NW
_C = 80
_NSC = _EPW // _C
_R = 4
_LOOP_END = (_NSC // _R) * _R
_RPT = 632
_RPT_LAST = _N - 15 * _RPT


def _segsum_sc(h, src, dst, zeros):
    mesh = plsc.VectorSubcoreMesh(core_axis_name="c", subcore_axis_name="s")

    @functools.partial(
        pl.kernel,
        out_type=jax.ShapeDtypeStruct((2, _N, _D), jnp.float32),
        mesh=mesh,
        scratch_types=(
            [pltpu.VMEM((_C,), jnp.int32) for _ in range(_R)]
            + [pltpu.VMEM((_C,), jnp.int32) for _ in range(_R)]
            + [pltpu.VMEM((_C, _D), jnp.float32) for _ in range(_R)]
            + [pltpu.VMEM_SHARED((_N, _D), jnp.float32)]
            + [pltpu.SemaphoreType.DMA for _ in range(2 * _R)]
        ),
    )
    def seg_kernel(h_hbm, src_hbm, dst_hbm, zero_hbm, out_hbm, *scratch):
        sidx = scratch[0:_R]
        didx = scratch[_R:2 * _R]
        rows = scratch[2 * _R:3 * _R]
        shared = scratch[3 * _R]
        gsem = scratch[3 * _R + 1:4 * _R + 1]
        ssem = scratch[4 * _R + 1:5 * _R + 1]
        bufs = tuple((sidx[r], didx[r], rows[r], gsem[r], ssem[r])
                     for r in range(_R))
        cid = lax.axis_index("c")
        sid = lax.axis_index("s")
        w = sid * 2 + cid

        def for_my_rows(fn):
            @pl.when(sid < 15)
            def _():
                fn(pl.ds(sid * _RPT, _RPT))

            @pl.when(sid == 15)
            def _():
                fn(pl.ds(15 * _RPT, _RPT_LAST))

        def init_rows(rslice):
            @pl.when(cid == 0)
            def _():
                pltpu.sync_copy(h_hbm.at[rslice], shared.at[rslice])

            @pl.when(cid != 0)
            def _():
                pltpu.sync_copy(zero_hbm.at[rslice], shared.at[rslice])

        for_my_rows(init_rows)
        plsc.subcore_barrier()

        ebase = w * _EPW

        def idx_copy(i, b):
            base = ebase + i * _C
            pltpu.sync_copy(src_hbm.at[pl.ds(base, _C)], b[0])
            pltpu.sync_copy(dst_hbm.at[pl.ds(base, _C)], b[1])

        def gstart(b):
            pltpu.async_copy(h_hbm.at[b[0]], b[2], b[3])

        def gwait(b):
            pltpu.make_async_copy(h_hbm.at[b[0]], b[2], b[3]).wait()

        def sstart(b):
            pltpu.async_copy(b[2], shared.at[b[1]], b[4], add=True)

        def swait(b):
            pltpu.make_async_copy(b[2], shared.at[b[1]], b[4]).wait()

        for j in range(_R - 1):
            idx_copy(j, bufs[j])
            gstart(bufs[j])

        @pl.loop(0, _LOOP_END, step=_R)
        def _(i):
            for k in range(_R):
                j = i + k
                b = bufs[k]
                prev = bufs[(k - 1) % _R]
                gwait(b)
                sstart(b)

                def refill(j=j, prev=prev):
                    idx_copy(j + _R - 1, prev)
                    gstart(prev)

                if k == 0:
                    @pl.when(i >= 1)
                    def _():
                        swait(prev)

                    @pl.when(i + _R - 1 < _NSC)
                    def _():
                        refill()
                else:
                    swait(prev)

                    @pl.when(j + _R - 1 < _NSC)
                    def _():
                        refill()

        for j in range(_LOOP_END, _NSC):
            b = bufs[j % _R]
            gwait(b)
            sstart(b)
            swait(bufs[(j - 1) % _R])
        swait(bufs[(_NSC - 1) % _R])

        plsc.subcore_barrier()
        for_my_rows(
            lambda rslice: pltpu.sync_copy(shared.at[rslice],
                                           out_hbm.at[cid].at[rslice]))

    return seg_kernel(h, src, dst, zeros)


def _layer_body(p_ref, w_ref, b_ref, g_ref, be_ref, o_ref):
    a = p_ref[0] + p_ref[1]
    pre = lax.dot_general(
        a, w_ref[...], (((1,), (0,)), ((), ())),
        precision=lax.Precision.HIGHEST,
        preferred_element_type=jnp.float32,
    ) + b_ref[...][None, :]
    mu = jnp.mean(pre, axis=0, keepdims=True)
    var = jnp.mean((pre - mu) ** 2, axis=0, keepdims=True)
    hn = (pre - mu) / jnp.sqrt(var + 1e-5) * g_ref[...][None, :] + be_ref[...][None, :]
    o_ref[...] = jnp.maximum(hn, 0.0)


def _tc_layer(parts, W, b, g, be):
    return pl.pallas_call(
        _layer_body,
        out_shape=jax.ShapeDtypeStruct((_N, _D), jnp.float32),
    )(parts, W, b, g, be)


def _head_body(h1_ref, bat1_ref, h2_ref, bat2_ref, wf_ref, bf_ref,
               wc1_ref, bc1_ref, wc2_ref, bc2_ref, o_ref):
    def pooled_emb(h_ref, bat_ref):
        bat = bat_ref[...]
        gids = lax.broadcasted_iota(jnp.int32, (_G, _N), 0)
        m = (bat[None, :] == gids).astype(jnp.float32)
        sums = lax.dot_general(
            m, h_ref[...], (((1,), (0,)), ((), ())),
            precision=lax.Precision.HIGHEST,
            preferred_element_type=jnp.float32,
        )
        cnt = jnp.sum(m, axis=1, keepdims=True)
        pooled = sums / jnp.maximum(cnt, 1.0)
        return lax.dot_general(
            pooled, wf_ref[...], (((1,), (0,)), ((), ())),
            precision=lax.Precision.HIGHEST,
            preferred_element_type=jnp.float32,
        ) + bf_ref[...][None, :]

    v1 = pooled_emb(h1_ref, bat1_ref)
    v2 = pooled_emb(h2_ref, bat2_ref)
    d = jnp.abs(v1 - v2)
    z = lax.dot_general(
        d, wc1_ref[...], (((1,), (0,)), ((), ())),
        precision=lax.Precision.HIGHEST,
        preferred_element_type=jnp.float32,
    ) + bc1_ref[...][None, :]
    z = jnp.maximum(z, 0.0)
    s = jnp.sum(z * wc2_ref[...][:, 0][None, :], axis=1, keepdims=True)
    s = s + bc2_ref[...][None, :]
    o_ref[...] = jax.nn.sigmoid(s)


def _head(h1, bat1, h2, bat2, Wf, bf, Wc1, bc1, Wc2, bc2):
    return pl.pallas_call(
        _head_body,
        out_shape=jax.ShapeDtypeStruct((_G, 1), jnp.float32),
    )(h1, bat1, h2, bat2, Wf, bf, Wc1, bc1, Wc2, bc2)


def kernel(x1, edge_index1, batch1, x2, edge_index2, batch2,
           W1, b1, g1, be1, W2, b2, g2, be2, W3, b3, g3, be3,
           Wf, bf, Wc1, bc1, Wc2, bc2):
    zeros = jnp.zeros((_N, _D), jnp.float32)
    layer_weights = ((W1, b1, g1, be1), (W2, b2, g2, be2), (W3, b3, g3, be3))

    def enc(x, ei):
        src, dst = ei[0], ei[1]
        h = x
        for (W, b, g, be) in layer_weights:
            parts = _segsum_sc(h, src, dst, zeros)
            h = _tc_layer(parts, W, b, g, be)
        return h

    h1 = enc(x1, edge_index1)
    h2 = enc(x2, edge_index2)
    return _head(h1, batch1, h2, batch2, Wf, bf, Wc1, bc1, Wc2, bc2)


# speedup vs baseline: 8.9104x; 1.1152x over previous
import functools

import jax
import jax.numpy as jnp
from jax import lax
from jax.experimental import pallas as pl
from jax.experimental.pallas import tpu as pltpu
from jax.experimental.pallas import tpu_sc as plsc

_N = 10000
_E = 320000
_D = 128
_G = 64
_NW = 32
_EPW = _E // _NW
_C = 128
_NFULL = _EPW // _C
_CT = _EPW - _NFULL * _C
_R = 3
_LOOP_END = (_NFULL // _R) * _R
_RPT = 632
_RPT_LAST = _N - 15 * _RPT


def _segsum_sc(h, src, dst, zeros):
    mesh = plsc.VectorSubcoreMesh(core_axis_name="c", subcore_axis_name="s")

    @functools.partial(
        pl.kernel,
        out_type=jax.ShapeDtypeStruct((2, _N, _D), jnp.float32),
        mesh=mesh,
        scratch_types=(
            [pltpu.VMEM((_C,), jnp.int32) for _ in range(_R)]
            + [pltpu.VMEM((_C,), jnp.int32) for _ in range(_R)]
            + [pltpu.VMEM((_C, _D), jnp.float32) for _ in range(_R)]
            + [pltpu.VMEM_SHARED((_N, _D), jnp.float32)]
            + [pltpu.SemaphoreType.DMA for _ in range(2 * _R)]
            + [pltpu.VMEM((_CT,), jnp.int32),
               pltpu.VMEM((_CT,), jnp.int32)]
        ),
    )
    def seg_kernel(h_hbm, src_hbm, dst_hbm, zero_hbm, out_hbm, *scratch):
        sidx = scratch[0:_R]
        didx = scratch[_R:2 * _R]
        rows = scratch[2 * _R:3 * _R]
        shared = scratch[3 * _R]
        gsem = scratch[3 * _R + 1:4 * _R + 1]
        ssem = scratch[4 * _R + 1:5 * _R + 1]
        sidx_t, didx_t = scratch[5 * _R + 1], scratch[5 * _R + 2]
        bufs = tuple((sidx[r], didx[r], rows[r], gsem[r], ssem[r])
                     for r in range(_R))
        cid = lax.axis_index("c")
        sid = lax.axis_index("s")
        w = sid * 2 + cid

        def for_my_rows(fn):
            @pl.when(sid < 15)
            def _():
                fn(pl.ds(sid * _RPT, _RPT))

            @pl.when(sid == 15)
            def _():
                fn(pl.ds(15 * _RPT, _RPT_LAST))

        def init_rows(rslice):
            @pl.when(cid == 0)
            def _():
                pltpu.sync_copy(h_hbm.at[rslice], shared.at[rslice])

            @pl.when(cid != 0)
            def _():
                pltpu.sync_copy(zero_hbm.at[rslice], shared.at[rslice])

        for_my_rows(init_rows)
        plsc.subcore_barrier()

        ebase = w * _EPW

        def idx_copy(i, b):
            base = ebase + i * _C
            pltpu.sync_copy(src_hbm.at[pl.ds(base, _C)], b[0])
            pltpu.sync_copy(dst_hbm.at[pl.ds(base, _C)], b[1])

        def gstart(b):
            pltpu.async_copy(h_hbm.at[b[0]], b[2], b[3])

        def gwait(b):
            pltpu.make_async_copy(h_hbm.at[b[0]], b[2], b[3]).wait()

        def sstart(b):
            pltpu.async_copy(b[2], shared.at[b[1]], b[4], add=True)

        def swait(b):
            pltpu.make_async_copy(b[2], shared.at[b[1]], b[4]).wait()

        for j in range(_R - 1):
            idx_copy(j, bufs[j])
            gstart(bufs[j])

        @pl.loop(0, _LOOP_END, step=_R)
        def _(i):
            for k in range(_R):
                j = i + k
                b = bufs[k]
                prev = bufs[(k - 1) % _R]
                gwait(b)
                sstart(b)

                def refill(j=j, prev=prev):
                    idx_copy(j + _R - 1, prev)
                    gstart(prev)

                if k == 0:
                    @pl.when(i >= 1)
                    def _():
                        swait(prev)

                    @pl.when(i + _R - 1 < _NFULL)
                    def _():
                        refill()
                else:
                    swait(prev)

                    @pl.when(j + _R - 1 < _NFULL)
                    def _():
                        refill()

        swait(bufs[(_NFULL - 1) % _R])

        tbase = ebase + _NFULL * _C
        rows_t = bufs[0][2].at[pl.ds(0, _CT)]
        pltpu.sync_copy(src_hbm.at[pl.ds(tbase, _CT)], sidx_t)
        pltpu.sync_copy(dst_hbm.at[pl.ds(tbase, _CT)], didx_t)
        pltpu.async_copy(h_hbm.at[sidx_t], rows_t, bufs[0][3]).wait()
        pltpu.sync_copy(rows_t, shared.at[didx_t], add=True)

        plsc.subcore_barrier()
        for_my_rows(
            lambda rslice: pltpu.sync_copy(shared.at[rslice],
                                           out_hbm.at[cid].at[rslice]))

    return seg_kernel(h, src, dst, zeros)


def _layer_body(p_ref, w_ref, b_ref, g_ref, be_ref, o_ref):
    a = p_ref[0] + p_ref[1]
    pre = lax.dot_general(
        a, w_ref[...], (((1,), (0,)), ((), ())),
        precision=lax.Precision.HIGHEST,
        preferred_element_type=jnp.float32,
    ) + b_ref[...][None, :]
    mu = jnp.mean(pre, axis=0, keepdims=True)
    var = jnp.mean((pre - mu) ** 2, axis=0, keepdims=True)
    hn = (pre - mu) / jnp.sqrt(var + 1e-5) * g_ref[...][None, :] + be_ref[...][None, :]
    o_ref[...] = jnp.maximum(hn, 0.0)


def _tc_layer(parts, W, b, g, be):
    return pl.pallas_call(
        _layer_body,
        out_shape=jax.ShapeDtypeStruct((_N, _D), jnp.float32),
    )(parts, W, b, g, be)


def _head_body(h1_ref, bat1_ref, h2_ref, bat2_ref, wf_ref, bf_ref,
               wc1_ref, bc1_ref, wc2_ref, bc2_ref, o_ref):
    def pooled_emb(h_ref, bat_ref):
        bat = bat_ref[...]
        gids = lax.broadcasted_iota(jnp.int32, (_G, _N), 0)
        m = (bat[None, :] == gids).astype(jnp.float32)
        sums = lax.dot_general(
            m, h_ref[...], (((1,), (0,)), ((), ())),
            precision=lax.Precision.HIGHEST,
            preferred_element_type=jnp.float32,
        )
        cnt = jnp.sum(m, axis=1, keepdims=True)
        pooled = sums / jnp.maximum(cnt, 1.0)
        return lax.dot_general(
            pooled, wf_ref[...], (((1,), (0,)), ((), ())),
            precision=lax.Precision.HIGHEST,
            preferred_element_type=jnp.float32,
        ) + bf_ref[...][None, :]

    v1 = pooled_emb(h1_ref, bat1_ref)
    v2 = pooled_emb(h2_ref, bat2_ref)
    d = jnp.abs(v1 - v2)
    z = lax.dot_general(
        d, wc1_ref[...], (((1,), (0,)), ((), ())),
        precision=lax.Precision.HIGHEST,
        preferred_element_type=jnp.float32,
    ) + bc1_ref[...][None, :]
    z = jnp.maximum(z, 0.0)
    s = jnp.sum(z * wc2_ref[...][:, 0][None, :], axis=1, keepdims=True)
    s = s + bc2_ref[...][None, :]
    o_ref[...] = jax.nn.sigmoid(s)


def _head(h1, bat1, h2, bat2, Wf, bf, Wc1, bc1, Wc2, bc2):
    return pl.pallas_call(
        _head_body,
        out_shape=jax.ShapeDtypeStruct((_G, 1), jnp.float32),
    )(h1, bat1, h2, bat2, Wf, bf, Wc1, bc1, Wc2, bc2)


def kernel(x1, edge_index1, batch1, x2, edge_index2, batch2,
           W1, b1, g1, be1, W2, b2, g2, be2, W3, b3, g3, be3,
           Wf, bf, Wc1, bc1, Wc2, bc2):
    zeros = jnp.zeros((_N, _D), jnp.float32)
    layer_weights = ((W1, b1, g1, be1), (W2, b2, g2, be2), (W3, b3, g3, be3))

    def enc(x, ei):
        src, dst = ei[0], ei[1]
        h = x
        for (W, b, g, be) in layer_weights:
            parts = _segsum_sc(h, src, dst, zeros)
            h = _tc_layer(parts, W, b, g, be)
        return h

    h1 = enc(x1, edge_index1)
    h2 = enc(x2, edge_index2)
    return _head(h1, batch1, h2, batch2, Wf, bf, Wc1, bc1, Wc2, bc2)
